# baseline (speedup 1.0000x reference)
import jax
import jax.numpy as jnp
from jax.experimental import pallas as pl

_N = 35
_E = 1190
_EP = 1280
_EB = 128
_NB = _EP // _EB


def _dot(a, b, prec=jax.lax.Precision.DEFAULT):
    return jax.lax.dot_general(a, b, (((1,), (0,)), ((), ())), precision=prec)


def _dotf32(a, b):
    return jax.lax.dot_general(a, b, (((1,), (0,)), ((), ())),
                               preferred_element_type=jnp.float32)


def _kern(src_ref, dst_ref, ea_ref,
          x_ref, W1_ref, root1_ref, bias1_ref,
          W2_ref, root2_ref, bias2_ref,
          W3_ref, root3_ref, bias3_ref,
          out_ref):
    f32 = jnp.float32
    bf16 = jnp.bfloat16
    src = src_ref[...]
    dst = dst_ref[...]

    dst_ohT = (jax.lax.broadcasted_iota(jnp.int32, (_N, _EP), 0)
               == dst).astype(f32)
    cnt = jnp.sum(dst_ohT, axis=1, keepdims=True)
    inv = 1.0 / jnp.maximum(cnt, 1.0)

    x = x_ref[...]
    wg1 = jnp.maximum(_dotf32(ea_ref[...], W1_ref[...]), 0.0)
    src_oh = (jax.lax.broadcasted_iota(jnp.int32, (_EP, _N), 1)
              == src).astype(f32)
    x_j = _dot(src_oh, x)
    agg = _dot(dst_ohT, x_j * wg1)
    h = _dot(x, root1_ref[...]) + agg * inv + bias1_ref[...]
    h = jnp.maximum(h, 0.0)

    for (W_ref, root_ref, bias_ref, cin, cout) in (
            (W2_ref, root2_ref, bias2_ref, 128, 128),
            (W3_ref, root3_ref, bias3_ref, 128, 64)):
        W = W_ref[...]
        h_in = h
        h16 = h_in.astype(bf16)
        hexp = jnp.concatenate(
            [jnp.broadcast_to(h16[:, i:i + 1], (_N, cout)) for i in range(cin)],
            axis=1)

        def make(off, W=W, hexp=hexp):
            a = ea_ref[pl.ds(off, _EB), :]
            wg = jnp.maximum(_dotf32(a, W), 0.0)
            s = src_ref[pl.ds(off, _EB), :]
            s_oh = (jax.lax.broadcasted_iota(jnp.int32, (_EB, _N), 1)
                    == s).astype(jnp.bfloat16)
            xje = _dotf32(s_oh, hexp)
            return wg, xje

        def consume(off, wg, xje, cin=cin, cout=cout):
            p = xje * wg
            width = cin * cout
            while width >= 4 * cout:
                q = width // 4
                p = ((p[:, :q] + p[:, q:2 * q])
                     + (p[:, 2 * q:3 * q] + p[:, 3 * q:width]))
                width = q
            while width > cout:
                half = width // 2
                p = p[:, :half] + p[:, half:width]
                width = half
            d = dst_ref[0:1, pl.ds(off, _EB)]
            d_ohT = (jax.lax.broadcasted_iota(jnp.int32, (_N, _EB), 0)
                     == d).astype(f32)
            return _dot(d_ohT, p)

        wg0, xje0 = make(0)

        def body(it, carry):
            agg, wg, xje = carry
            wg_n, xje_n = make((it + 1) * _EB)
            agg = agg + consume(it * _EB, wg, xje)
            return (agg, wg_n, xje_n)

        agg, wg_l, xje_l = jax.lax.fori_loop(
            0, _NB - 1, body, (jnp.zeros((_N, cout), f32), wg0, xje0))
        agg = agg + consume((_NB - 1) * _EB, wg_l, xje_l)
        h = _dot(h_in, root_ref[...]) + agg * inv + bias_ref[...]
        h = jnp.maximum(h, 0.0)

    cols = []
    for j in range(_N):
        d = jnp.sum(jnp.abs(h - h[j:j + 1, :]), axis=1, keepdims=True)
        cols.append(d)
    out_ref[...] = jnp.concatenate(cols, axis=1)


@jax.jit
def kernel(x, edge_attr, edge_index, W1, b1, root1, bias1,
           W2, b2, root2, bias2, W3, b3, root3, bias3):
    f32 = jnp.float32
    bf16 = jnp.bfloat16
    src = jnp.full((_EP, 1), _N, jnp.int32).at[:_E, 0].set(edge_index[0])
    dst = jnp.full((1, _EP), _N, jnp.int32).at[0, :_E].set(edge_index[1])
    ea = (jnp.ones((_EP, 7), f32).at[:_E, :6].set(edge_attr)
          .at[_E:, :6].set(0.0).astype(bf16))
    W1b = jnp.concatenate([W1, b1.reshape(1, -1)], axis=0).astype(bf16)
    W2b = jnp.concatenate([W2, b2.reshape(1, -1)], axis=0).astype(bf16)
    W3b = jnp.concatenate([W3, b3.reshape(1, -1)], axis=0).astype(bf16)

    return pl.pallas_call(
        _kern,
        out_shape=jax.ShapeDtypeStruct((_N, _N), f32),
    )(src, dst, ea,
      x, W1b, root1, bias1.reshape(1, -1),
      W2b, root2, bias2.reshape(1, -1),
      W3b, root3, bias3.reshape(1, -1))


# speedup vs baseline: 1.0494x; 1.0494x over previous
import jax
import jax.numpy as jnp
from jax.experimental import pallas as pl

_N = 35
_E = 1190
_EP = 1280
_EB = 128
_NB = _EP // _EB


def _dot(a, b):
    return jax.lax.dot_general(a, b, (((1,), (0,)), ((), ())),
                               preferred_element_type=jnp.float32)


def _kern(src_ref, dst_ref, ea_ref,
          x_ref, W1_ref, root1_ref, bias1_ref,
          W2_ref, root2_ref, bias2_ref,
          W3_ref, root3_ref, bias3_ref,
          out_ref):
    f32 = jnp.float32
    src = src_ref[...]
    dst = dst_ref[...]

    dst_ohT = (jax.lax.broadcasted_iota(jnp.int32, (_N, _EP), 0)
               == dst).astype(f32)
    cnt = jnp.sum(dst_ohT, axis=1, keepdims=True)
    inv = 1.0 / jnp.maximum(cnt, 1.0)

    x = x_ref[...]
    wg1 = jnp.maximum(_dot(ea_ref[...], W1_ref[...]), 0.0)
    src_oh = (jax.lax.broadcasted_iota(jnp.int32, (_EP, _N), 1)
              == src).astype(f32)
    x_j = _dot(src_oh, x)
    agg = _dot(dst_ohT, x_j * wg1)
    h = _dot(x, root1_ref[...]) + agg * inv + bias1_ref[...]
    h = jnp.maximum(h, 0.0)

    for (W_ref, root_ref, bias_ref, cin, cout) in (
            (W2_ref, root2_ref, bias2_ref, 128, 128),
            (W3_ref, root3_ref, bias3_ref, 128, 64)):
        h_in = h
        hexp = jnp.concatenate(
            [jnp.broadcast_to(h_in[:, i:i + 1], (_N, cout)) for i in range(cin)],
            axis=1)
        rhs = jnp.concatenate([W_ref[...], hexp], axis=0)

        def one_block(off, rhs=rhs, h_in=h_in, cin=cin, cout=cout):
            a = ea_ref[pl.ds(off, _EB), :]
            s = src_ref[pl.ds(off, _EB), :]
            s_oh = (jax.lax.broadcasted_iota(jnp.int32, (_EB, _N), 1)
                    == s).astype(f32)
            lhs = jnp.concatenate([a, s_oh], axis=1)
            S = _dot(lhs, rhs)
            xj = _dot(s_oh, h_in)
            xje = jnp.concatenate(
                [jnp.broadcast_to(xj[:, i:i + 1], (_EB, cout))
                 for i in range(cin)], axis=1)
            p = xje * jnp.maximum(S - xje, 0.0)
            width = cin * cout
            while width >= 4 * cout:
                q = width // 4
                p = ((p[:, :q] + p[:, q:2 * q])
                     + (p[:, 2 * q:3 * q] + p[:, 3 * q:width]))
                width = q
            while width > cout:
                half = width // 2
                p = p[:, :half] + p[:, half:width]
                width = half
            d = dst_ref[0:1, pl.ds(off, _EB)]
            d_ohT = (jax.lax.broadcasted_iota(jnp.int32, (_N, _EB), 0)
                     == d).astype(f32)
            return _dot(d_ohT, p)

        def body(it, agg):
            off = it * (2 * _EB)
            return agg + one_block(off) + one_block(off + _EB)

        agg = jax.lax.fori_loop(0, _NB // 2, body,
                                jnp.zeros((_N, cout), f32))
        h = _dot(h_in, root_ref[...]) + agg * inv + bias_ref[...]
        h = jnp.maximum(h, 0.0)

    cols = []
    for j in range(_N):
        d = jnp.sum(jnp.abs(h - h[j:j + 1, :]), axis=1, keepdims=True)
        cols.append(d)
    out_ref[...] = jnp.concatenate(cols, axis=1)


@jax.jit
def kernel(x, edge_attr, edge_index, W1, b1, root1, bias1,
           W2, b2, root2, bias2, W3, b3, root3, bias3):
    f32 = jnp.float32
    src = jnp.full((_EP, 1), _N, jnp.int32).at[:_E, 0].set(edge_index[0])
    dst = jnp.full((1, _EP), _N, jnp.int32).at[0, :_E].set(edge_index[1])
    ea = (jnp.zeros((_EP, 8), f32).at[:_E, :6].set(edge_attr)
          .at[:, 6].set(1.0))
    zrow = jnp.zeros((1, 1), f32)

    def wb(Wm, bv):
        z = jnp.zeros((1, bv.shape[0]), f32)
        return jnp.concatenate([Wm, bv.reshape(1, -1), z], axis=0)

    W1b = wb(W1, b1)
    W2b = wb(W2, b2)
    W3b = wb(W3, b3)

    return pl.pallas_call(
        _kern,
        out_shape=jax.ShapeDtypeStruct((_N, _N), f32),
    )(src, dst, ea,
      x, W1b, root1, bias1.reshape(1, -1),
      W2b, root2, bias2.reshape(1, -1),
      W3b, root3, bias3.reshape(1, -1))


# speedup vs baseline: 1.8353x; 1.7489x over previous
import jax
import jax.numpy as jnp
from jax.experimental import pallas as pl

_N = 35
_E = 1190
_EP = 1280
_EB = 128
_NB = _EP // _EB


def _dot(a, b, prec=jax.lax.Precision.DEFAULT):
    return jax.lax.dot_general(a, b, (((1,), (0,)), ((), ())), precision=prec)


def _dotf32(a, b):
    return jax.lax.dot_general(a.astype(jnp.bfloat16), b.astype(jnp.bfloat16),
                               (((1,), (0,)), ((), ())),
                               preferred_element_type=jnp.float32)


def _kern(src_ref, dst_ref, ea_ref,
          x_ref, W1_ref, root1_ref, bias1_ref,
          W2_ref, root2_ref, bias2_ref,
          W3_ref, root3_ref, bias3_ref,
          out_ref):
    f32 = jnp.float32
    src = src_ref[...]
    dst = dst_ref[...]

    dst_ohT = (jax.lax.broadcasted_iota(jnp.int32, (_N, _EP), 0)
               == dst).astype(f32)
    cnt = jnp.sum(dst_ohT, axis=1, keepdims=True)
    inv = 1.0 / jnp.maximum(cnt, 1.0)

    x = x_ref[...]
    wg1 = jnp.maximum(_dotf32(ea_ref[...], W1_ref[...]), 0.0)
    src_oh = (jax.lax.broadcasted_iota(jnp.int32, (_EP, _N), 1)
              == src).astype(f32)
    x_j = _dot(src_oh, x)
    agg = _dot(dst_ohT, x_j * wg1)
    h = _dot(x, root1_ref[...]) + agg * inv + bias1_ref[...]
    h = jnp.maximum(h, 0.0)

    for (W_ref, root_ref, bias_ref, cin, cout) in (
            (W2_ref, root2_ref, bias2_ref, 128, 128),
            (W3_ref, root3_ref, bias3_ref, 128, 64)):
        W = W_ref[...]
        h_in = h
        hexp = jnp.concatenate(
            [jnp.broadcast_to(h_in[:, i:i + 1], (_N, cout)) for i in range(cin)],
            axis=1)

        def one_block(off, W=W, hexp=hexp, cin=cin, cout=cout):
            a = ea_ref[pl.ds(off, _EB), :]
            wg = jnp.maximum(_dotf32(a, W), 0.0)
            s = src_ref[pl.ds(off, _EB), :]
            s_oh = (jax.lax.broadcasted_iota(jnp.int32, (_EB, _N), 1)
                    == s).astype(f32)
            xje = _dotf32(s_oh, hexp)
            p = xje * wg
            width = cin * cout
            while width >= 4 * cout:
                q = width // 4
                p = ((p[:, :q] + p[:, q:2 * q])
                     + (p[:, 2 * q:3 * q] + p[:, 3 * q:width]))
                width = q
            while width > cout:
                half = width // 2
                p = p[:, :half] + p[:, half:width]
                width = half
            d = dst_ref[0:1, pl.ds(off, _EB)]
            d_ohT = (jax.lax.broadcasted_iota(jnp.int32, (_N, _EB), 0)
                     == d).astype(f32)
            return _dot(d_ohT, p)

        def body(it, agg):
            off = it * (2 * _EB)
            return agg + one_block(off) + one_block(off + _EB)

        agg = jax.lax.fori_loop(0, _NB // 2, body,
                                jnp.zeros((_N, cout), f32))
        h = _dot(h_in, root_ref[...]) + agg * inv + bias_ref[...]
        h = jnp.maximum(h, 0.0)

    cols = []
    for j in range(_N):
        d = jnp.sum(jnp.abs(h - h[j:j + 1, :]), axis=1, keepdims=True)
        cols.append(d)
    out_ref[...] = jnp.concatenate(cols, axis=1)


@jax.jit
def kernel(x, edge_attr, edge_index, W1, b1, root1, bias1,
           W2, b2, root2, bias2, W3, b3, root3, bias3):
    f32 = jnp.float32
    src = jnp.full((_EP, 1), _N, jnp.int32).at[:_E, 0].set(edge_index[0])
    dst = jnp.full((1, _EP), _N, jnp.int32).at[0, :_E].set(edge_index[1])
    ea = (jnp.ones((_EP, 7), f32).at[:_E, :6].set(edge_attr)
          .at[_E:, :6].set(0.0))
    W1b = jnp.concatenate([W1, b1.reshape(1, -1)], axis=0)
    W2b = jnp.concatenate([W2, b2.reshape(1, -1)], axis=0)
    W3b = jnp.concatenate([W3, b3.reshape(1, -1)], axis=0)

    return pl.pallas_call(
        _kern,
        out_shape=jax.ShapeDtypeStruct((_N, _N), f32),
    )(src, dst, ea,
      x, W1b, root1, bias1.reshape(1, -1),
      W2b, root2, bias2.reshape(1, -1),
      W3b, root3, bias3.reshape(1, -1))
